# baseline (speedup 1.0000x reference)
import jax
import jax.numpy as jnp
from jax.experimental import pallas as pl
from jax.experimental.pallas import tpu as pltpu

_BLOCK = 4096
_KSPLIT = 4


def _dedicom_body(row_ref, col_ref, g_ref, lv_ref, out1_ref, out2_ref):
    rowb = row_ref[...].astype(jnp.bfloat16)
    colb = col_ref[...].astype(jnp.bfloat16)
    g = g_ref[...]
    lv = lv_ref[...]
    k_rel = lv.shape[0]
    d = g.shape[0]
    m_ks = [((lv[k][:, None] * g) * lv[k][None, :]).astype(jnp.bfloat16)
            for k in range(k_rel)]
    recs = []
    for k in range(_KSPLIT):
        left = jnp.dot(rowb, m_ks[k], preferred_element_type=jnp.float32)
        t = left.astype(jnp.bfloat16) * colb
        recs.append(jnp.sum(t, axis=1, dtype=jnp.bfloat16))
    scores = jnp.stack(recs, axis=0).astype(jnp.float32)
    out1_ref[...] = jax.nn.sigmoid(scores)
    ts = []
    for k in range(_KSPLIT, k_rel):
        left = jnp.dot(rowb, m_ks[k], preferred_element_type=jnp.float32)
        ts.append(left.astype(jnp.bfloat16) * colb)
    t_all = jnp.concatenate(ts, axis=1)
    n_seg = k_rel - _KSPLIT
    m_idx = jax.lax.broadcasted_iota(jnp.int32, (n_seg * d, n_seg), 0)
    k_idx = jax.lax.broadcasted_iota(jnp.int32, (n_seg * d, n_seg), 1)
    seg = (m_idx // d == k_idx).astype(jnp.bfloat16)
    rec2 = jnp.dot(t_all, seg, preferred_element_type=jnp.float32)
    out2_ref[...] = jax.nn.sigmoid(rec2)


def kernel(inputs_row, inputs_col, global_interaction, local_variation):
    n, d = inputs_row.shape
    k_rel = local_variation.shape[0]
    grid = (pl.cdiv(n, _BLOCK),)
    out1, out2 = pl.pallas_call(
        _dedicom_body,
        grid=grid,
        in_specs=[
            pl.BlockSpec((_BLOCK, d), lambda i: (i, 0)),
            pl.BlockSpec((_BLOCK, d), lambda i: (i, 0)),
            pl.BlockSpec((d, d), lambda i: (0, 0)),
            pl.BlockSpec((k_rel, d), lambda i: (0, 0)),
        ],
        out_specs=[
            pl.BlockSpec((_KSPLIT, _BLOCK), lambda i: (0, i)),
            pl.BlockSpec((_BLOCK, k_rel - _KSPLIT), lambda i: (i, 0)),
        ],
        out_shape=[
            jax.ShapeDtypeStruct((_KSPLIT, n), jnp.float32),
            jax.ShapeDtypeStruct((n, k_rel - _KSPLIT), jnp.float32),
        ],
        compiler_params=pltpu.CompilerParams(
            dimension_semantics=("parallel",),
        ),
        name="dedicom_decoder",
    )(inputs_row, inputs_col, global_interaction, local_variation)
    return jnp.concatenate([out1, out2.T], axis=0)


# speedup vs baseline: 1.0334x; 1.0334x over previous
import jax
import jax.numpy as jnp
from jax.experimental import pallas as pl
from jax.experimental.pallas import tpu as pltpu

_BLOCK = 8192
_KSPLIT = 4


def _dedicom_body(row_ref, col_ref, g_ref, lv_ref, out1_ref, out2_ref):
    rowb = row_ref[...].astype(jnp.bfloat16)
    colb = col_ref[...].astype(jnp.bfloat16)
    g = g_ref[...]
    lv = lv_ref[...]
    k_rel = lv.shape[0]
    d = g.shape[0]
    m_ks = [((lv[k][:, None] * g) * lv[k][None, :]).astype(jnp.bfloat16)
            for k in range(k_rel)]
    recs = []
    for k in range(_KSPLIT):
        left = jnp.dot(rowb, m_ks[k], preferred_element_type=jnp.float32)
        t = left.astype(jnp.bfloat16) * colb
        recs.append(jnp.sum(t, axis=1, dtype=jnp.bfloat16))
    scores = jnp.stack(recs, axis=0).astype(jnp.float32)
    out1_ref[...] = jax.nn.sigmoid(scores)
    ts = []
    for k in range(_KSPLIT, k_rel):
        left = jnp.dot(rowb, m_ks[k], preferred_element_type=jnp.float32)
        ts.append(left.astype(jnp.bfloat16) * colb)
    t_all = jnp.concatenate(ts, axis=1)
    n_seg = k_rel - _KSPLIT
    m_idx = jax.lax.broadcasted_iota(jnp.int32, (n_seg * d, n_seg), 0)
    k_idx = jax.lax.broadcasted_iota(jnp.int32, (n_seg * d, n_seg), 1)
    seg = (m_idx // d == k_idx).astype(jnp.bfloat16)
    rec2 = jnp.dot(t_all, seg, preferred_element_type=jnp.float32)
    out2_ref[...] = jax.nn.sigmoid(rec2)


def kernel(inputs_row, inputs_col, global_interaction, local_variation):
    n, d = inputs_row.shape
    k_rel = local_variation.shape[0]
    grid = (pl.cdiv(n, _BLOCK),)
    out1, out2 = pl.pallas_call(
        _dedicom_body,
        grid=grid,
        in_specs=[
            pl.BlockSpec((_BLOCK, d), lambda i: (i, 0)),
            pl.BlockSpec((_BLOCK, d), lambda i: (i, 0)),
            pl.BlockSpec((d, d), lambda i: (0, 0)),
            pl.BlockSpec((k_rel, d), lambda i: (0, 0)),
        ],
        out_specs=[
            pl.BlockSpec((_KSPLIT, _BLOCK), lambda i: (0, i)),
            pl.BlockSpec((_BLOCK, k_rel - _KSPLIT), lambda i: (i, 0)),
        ],
        out_shape=[
            jax.ShapeDtypeStruct((_KSPLIT, n), jnp.float32),
            jax.ShapeDtypeStruct((n, k_rel - _KSPLIT), jnp.float32),
        ],
        compiler_params=pltpu.CompilerParams(
            dimension_semantics=("parallel",),
        ),
        name="dedicom_decoder",
    )(inputs_row, inputs_col, global_interaction, local_variation)
    return jnp.concatenate([out1, out2.T], axis=0)


# speedup vs baseline: 1.0342x; 1.0008x over previous
import jax
import jax.numpy as jnp
from jax.experimental import pallas as pl
from jax.experimental.pallas import tpu as pltpu

_BLOCK = 8192


def _dedicom_body(row_ref, col_ref, g_ref, lv_ref, out_ref):
    row = row_ref[...]
    col = col_ref[...]
    g = g_ref[...]
    k_rel = lv_ref.shape[0]
    recs = []
    for k in range(k_rel):
        dk = lv_ref[k, :]
        left = jnp.dot(row * dk[None, :], g,
                       preferred_element_type=jnp.float32)
        recs.append(jnp.sum(left * (col * dk[None, :]), axis=1))
    scores = jnp.stack(recs, axis=0)
    out_ref[...] = jax.nn.sigmoid(scores)


def kernel(inputs_row, inputs_col, global_interaction, local_variation):
    n, d = inputs_row.shape
    k_rel = local_variation.shape[0]
    grid = (pl.cdiv(n, _BLOCK),)
    return pl.pallas_call(
        _dedicom_body,
        grid=grid,
        in_specs=[
            pl.BlockSpec((_BLOCK, d), lambda i: (i, 0)),
            pl.BlockSpec((_BLOCK, d), lambda i: (i, 0)),
            pl.BlockSpec((d, d), lambda i: (0, 0)),
            pl.BlockSpec((k_rel, d), lambda i: (0, 0)),
        ],
        out_specs=pl.BlockSpec((k_rel, _BLOCK), lambda i: (0, i)),
        out_shape=jax.ShapeDtypeStruct((k_rel, n), jnp.float32),
        compiler_params=pltpu.CompilerParams(
            dimension_semantics=("parallel",),
        ),
        name="dedicom_decoder",
    )(inputs_row, inputs_col, global_interaction, local_variation)


# speedup vs baseline: 1.3199x; 1.2762x over previous
import jax
import jax.numpy as jnp
from jax.experimental import pallas as pl
from jax.experimental.pallas import tpu as pltpu

_BLOCK = 8192
_KSPLIT = 4


def _dedicom_body(row_ref, col_ref, g_ref, lv_ref, out_ref):
    rowb = row_ref[...].astype(jnp.bfloat16)
    colb = col_ref[...].astype(jnp.bfloat16)
    g = g_ref[...]
    lv = lv_ref[...]
    k_rel = lv.shape[0]
    d = g.shape[0]
    m_ks = [((lv[k][:, None] * g) * lv[k][None, :]).astype(jnp.bfloat16)
            for k in range(k_rel)]
    recs = []
    for k in range(_KSPLIT):
        left = jnp.dot(rowb, m_ks[k], preferred_element_type=jnp.float32)
        t = left.astype(jnp.bfloat16) * colb
        recs.append(jnp.sum(t, axis=1, dtype=jnp.bfloat16))
    scores = jnp.stack(recs, axis=0).astype(jnp.float32)
    out_ref[0:_KSPLIT, :] = jax.nn.sigmoid(scores)
    ts = []
    for k in range(_KSPLIT, k_rel):
        left = jnp.dot(rowb, m_ks[k], preferred_element_type=jnp.float32)
        ts.append(left.astype(jnp.bfloat16) * colb)
    t_all = jnp.concatenate(ts, axis=1)
    n_seg = k_rel - _KSPLIT
    m_idx = jax.lax.broadcasted_iota(jnp.int32, (n_seg * d, n_seg), 0)
    k_idx = jax.lax.broadcasted_iota(jnp.int32, (n_seg * d, n_seg), 1)
    seg = (m_idx // d == k_idx).astype(jnp.bfloat16)
    rec2 = jnp.dot(t_all, seg, preferred_element_type=jnp.float32)
    out_ref[_KSPLIT:, :] = jax.nn.sigmoid(rec2.T)


def kernel(inputs_row, inputs_col, global_interaction, local_variation):
    n, d = inputs_row.shape
    k_rel = local_variation.shape[0]
    grid = (pl.cdiv(n, _BLOCK),)
    return pl.pallas_call(
        _dedicom_body,
        grid=grid,
        in_specs=[
            pl.BlockSpec((_BLOCK, d), lambda i: (i, 0)),
            pl.BlockSpec((_BLOCK, d), lambda i: (i, 0)),
            pl.BlockSpec((d, d), lambda i: (0, 0)),
            pl.BlockSpec((k_rel, d), lambda i: (0, 0)),
        ],
        out_specs=pl.BlockSpec((k_rel, _BLOCK), lambda i: (0, i)),
        out_shape=jax.ShapeDtypeStruct((k_rel, n), jnp.float32),
        compiler_params=pltpu.CompilerParams(
            dimension_semantics=("parallel",),
        ),
        name="dedicom_decoder",
    )(inputs_row, inputs_col, global_interaction, local_variation)


# speedup vs baseline: 1.3466x; 1.0202x over previous
import jax
import jax.numpy as jnp
from jax.experimental import pallas as pl
from jax.experimental.pallas import tpu as pltpu

_BLOCK = 12288
_KSPLIT = 4


def _dedicom_body(row_ref, col_ref, g_ref, lv_ref, out_ref):
    rowb = row_ref[...].astype(jnp.bfloat16)
    colb = col_ref[...].astype(jnp.bfloat16)
    g = g_ref[...]
    lv = lv_ref[...]
    k_rel = lv.shape[0]
    d = g.shape[0]
    m_ks = [((lv[k][:, None] * g) * lv[k][None, :]).astype(jnp.bfloat16)
            for k in range(k_rel)]
    recs = []
    for k in range(_KSPLIT):
        left = jnp.dot(rowb, m_ks[k], preferred_element_type=jnp.float32)
        t = left.astype(jnp.bfloat16) * colb
        recs.append(jnp.sum(t, axis=1, dtype=jnp.bfloat16))
    scores = jnp.stack(recs, axis=0).astype(jnp.float32)
    out_ref[0:_KSPLIT, :] = jax.nn.sigmoid(scores)
    ts = []
    for k in range(_KSPLIT, k_rel):
        left = jnp.dot(rowb, m_ks[k], preferred_element_type=jnp.float32)
        ts.append(left.astype(jnp.bfloat16) * colb)
    t_all = jnp.concatenate(ts, axis=1)
    n_seg = k_rel - _KSPLIT
    m_idx = jax.lax.broadcasted_iota(jnp.int32, (n_seg * d, n_seg), 0)
    k_idx = jax.lax.broadcasted_iota(jnp.int32, (n_seg * d, n_seg), 1)
    seg = (m_idx // d == k_idx).astype(jnp.bfloat16)
    rec2 = jnp.dot(t_all, seg, preferred_element_type=jnp.float32)
    out_ref[_KSPLIT:, :] = jax.nn.sigmoid(rec2.T)


def kernel(inputs_row, inputs_col, global_interaction, local_variation):
    n, d = inputs_row.shape
    k_rel = local_variation.shape[0]
    grid = (pl.cdiv(n, _BLOCK),)
    return pl.pallas_call(
        _dedicom_body,
        grid=grid,
        in_specs=[
            pl.BlockSpec((_BLOCK, d), lambda i: (i, 0)),
            pl.BlockSpec((_BLOCK, d), lambda i: (i, 0)),
            pl.BlockSpec((d, d), lambda i: (0, 0)),
            pl.BlockSpec((k_rel, d), lambda i: (0, 0)),
        ],
        out_specs=pl.BlockSpec((k_rel, _BLOCK), lambda i: (0, i)),
        out_shape=jax.ShapeDtypeStruct((k_rel, n), jnp.float32),
        compiler_params=pltpu.CompilerParams(
            dimension_semantics=("parallel",),
        ),
        name="dedicom_decoder",
    )(inputs_row, inputs_col, global_interaction, local_variation)
